# Initial kernel scaffold; baseline (speedup 1.0000x reference)
import jax
import jax.numpy as jnp
from jax.experimental import pallas as pl


def kernel(inputs, labels, moco_inputs, emb_cq, label_cq, age_cq):
    raise NotImplementedError("write your pallas kernel here")


# speedup vs baseline: 2.0070x; 2.0070x over previous
import functools

import jax
import jax.numpy as jnp
from jax.experimental import pallas as pl
from jax.experimental.pallas import tpu as pltpu

_TEMP = 0.1
_TINY = 1.1754944e-38
_EPS = 1e-12


def _double_normalize(v):
    n = jnp.sqrt(jnp.sum(v * v, axis=1, keepdims=True))
    v = v / jnp.maximum(n, _EPS)
    n2 = jnp.sqrt(jnp.sum(v * v, axis=1, keepdims=True))
    return v / jnp.maximum(n2, _EPS)


def _loss_kernel(TM, B, x_ref, labr_ref, laba_ref, m_ref, out_ref,
                 rn_ref, sr_ref, acc_ref):
    i = pl.program_id(0)
    nsteps = pl.num_programs(0)

    @pl.when(i == 0)
    def _init():
        rn = _double_normalize(m_ref[...])
        rn_ref[...] = rn
        rsq = rn * rn
        ones = jnp.ones((1, rsq.shape[1]), jnp.float32)
        sr_ref[...] = jax.lax.dot_general(
            ones, rsq, (((1,), (1,)), ((), ())),
            preferred_element_type=jnp.float32)
        acc_ref[0] = 0.0
        acc_ref[1] = 0.0

    xn = _double_normalize(x_ref[...])
    rn = rn_ref[...]
    sim = jax.lax.dot_general(
        xn, rn, (((1,), (1,)), ((), ())),
        preferred_element_type=jnp.float32)
    sx = jnp.sum(xn * xn, axis=1, keepdims=True)
    d2 = sx + sr_ref[...] - 2.0 * sim
    dist = jnp.sqrt(jnp.maximum(d2, 0.0))

    pos = labr_ref[...] == laba_ref[...]
    col = jax.lax.broadcasted_iota(jnp.int32, (TM, B), 1)
    inf = jnp.inf

    pdist = jnp.where(pos, dist, -inf)
    pbest = jnp.max(pdist, axis=1, keepdims=True)
    pidx = jnp.min(jnp.where(pdist == pbest, col, B), axis=1, keepdims=True)
    pos_sim = jnp.sum(jnp.where(col == pidx, sim, 0.0), axis=1, keepdims=True)

    ndist = jnp.where(pos, inf, dist)
    nbest = jnp.min(ndist, axis=1, keepdims=True)
    nidx = jnp.min(jnp.where(ndist == nbest, col, B), axis=1, keepdims=True)
    neg_sim = jnp.sum(jnp.where(col == nidx, sim, 0.0), axis=1, keepdims=True)

    valid = nbest < inf

    p = pos_sim / _TEMP
    n = neg_sim / _TEMP
    mx = jnp.maximum(p, n)
    num = jnp.exp(p - mx)
    den = jnp.exp(n - mx) + num
    losses = -jnp.log(num / den + _TINY)

    acc_ref[0] += jnp.sum(jnp.where(valid, losses, 0.0))
    acc_ref[1] += jnp.sum(jnp.where(valid, 1.0, 0.0))

    @pl.when(i == nsteps - 1)
    def _fin():
        loss = acc_ref[0] / jnp.maximum(acc_ref[1], 1.0)
        out_ref[...] = jnp.full((1, 1), loss, jnp.float32)


def kernel(inputs, labels, moco_inputs, emb_cq, label_cq, age_cq):
    B, F = inputs.shape
    TM = 128
    lab_col = labels.reshape(B, 1)
    lab_row = labels.reshape(1, B)
    out = pl.pallas_call(
        functools.partial(_loss_kernel, TM, B),
        grid=(B // TM,),
        in_specs=[
            pl.BlockSpec((TM, F), lambda i: (i, 0)),
            pl.BlockSpec((TM, 1), lambda i: (i, 0)),
            pl.BlockSpec((1, B), lambda i: (0, 0)),
            pl.BlockSpec((B, F), lambda i: (0, 0)),
        ],
        out_specs=pl.BlockSpec((1, 1), lambda i: (0, 0)),
        out_shape=jax.ShapeDtypeStruct((1, 1), jnp.float32),
        scratch_shapes=[
            pltpu.VMEM((B, F), jnp.float32),
            pltpu.VMEM((1, B), jnp.float32),
            pltpu.SMEM((2,), jnp.float32),
        ],
    )(inputs, lab_col, lab_row, moco_inputs)
    return out[0, 0]


# speedup vs baseline: 6.8769x; 3.4265x over previous
import functools

import jax
import jax.numpy as jnp
from jax.experimental import pallas as pl
from jax.experimental.pallas import tpu as pltpu

_TEMP = 0.1
_TINY = 1.1754944e-38
_EPS = 1e-12


def _double_normalize(v):
    n = jnp.sqrt(jnp.sum(v * v, axis=1, keepdims=True))
    v = v / jnp.maximum(n, _EPS)
    n2 = jnp.sqrt(jnp.sum(v * v, axis=1, keepdims=True))
    return v / jnp.maximum(n2, _EPS)


def _loss_kernel(x_ref, labr_ref, laba_ref, m_ref, out_ref, rn_ref, acc_ref):
    i = pl.program_id(0)
    nsteps = pl.num_programs(0)

    @pl.when(i == 0)
    def _init():
        rn_ref[...] = _double_normalize(m_ref[...])
        acc_ref[0] = 0.0
        acc_ref[1] = 0.0

    xn = _double_normalize(x_ref[...])
    sim = jax.lax.dot_general(
        xn, rn_ref[...], (((1,), (1,)), ((), ())),
        preferred_element_type=jnp.float32)

    pos = labr_ref[...] == laba_ref[...]
    inf = jnp.inf

    pos_sim = jnp.min(jnp.where(pos, sim, inf), axis=1, keepdims=True)
    neg_sim = jnp.max(jnp.where(pos, -inf, sim), axis=1, keepdims=True)

    valid = neg_sim > -inf
    neg_sim = jnp.where(valid, neg_sim, 0.0)

    p = pos_sim / _TEMP
    n = neg_sim / _TEMP
    mx = jnp.maximum(p, n)
    num = jnp.exp(p - mx)
    den = jnp.exp(n - mx) + num
    losses = -jnp.log(num / den + _TINY)

    acc_ref[0] += jnp.sum(jnp.where(valid, losses, 0.0))
    acc_ref[1] += jnp.sum(jnp.where(valid, 1.0, 0.0))

    @pl.when(i == nsteps - 1)
    def _fin():
        loss = acc_ref[0] / jnp.maximum(acc_ref[1], 1.0)
        out_ref[...] = jnp.full((1, 1), loss, jnp.float32)


def kernel(inputs, labels, moco_inputs, emb_cq, label_cq, age_cq):
    B, F = inputs.shape
    TM = 256
    lab_col = labels.reshape(B, 1)
    lab_row = labels.reshape(1, B)
    out = pl.pallas_call(
        _loss_kernel,
        grid=(B // TM,),
        in_specs=[
            pl.BlockSpec((TM, F), lambda i: (i, 0)),
            pl.BlockSpec((TM, 1), lambda i: (i, 0)),
            pl.BlockSpec((1, B), lambda i: (0, 0)),
            pl.BlockSpec((B, F), lambda i: (0, 0)),
        ],
        out_specs=pl.BlockSpec((1, 1), lambda i: (0, 0)),
        out_shape=jax.ShapeDtypeStruct((1, 1), jnp.float32),
        scratch_shapes=[
            pltpu.VMEM((B, F), jnp.float32),
            pltpu.SMEM((2,), jnp.float32),
        ],
    )(inputs, lab_col, lab_row, moco_inputs)
    return out[0, 0]
